# baseline (speedup 1.0000x reference)
import functools

import jax
import jax.numpy as jnp
from jax import lax
from jax.experimental import pallas as pl
from jax.experimental.pallas import tpu as pltpu
from jax.experimental.pallas import tpu_sc as plsc

_NUM_CORES = 2
_NUM_SUBCORES = 16
_NUM_WORKERS = _NUM_CORES * _NUM_SUBCORES
_LANES = 16
_SUB = 8
_LANE_T = 128


@functools.cache
def _build(batch, vocab, dim):
  b_per_w = batch // _NUM_WORKERS
  n_lane_t = b_per_w // _LANE_T
  n_sub_t = dim // _SUB
  mesh = plsc.VectorSubcoreMesh(core_axis_name="c", subcore_axis_name="s")

  @functools.partial(
      pl.kernel,
      mesh=mesh,
      out_type=jax.ShapeDtypeStruct(
          (n_sub_t, batch // _LANE_T, _SUB, _LANE_T), jnp.float32
      ),
      scratch_types=[
          pltpu.VMEM((b_per_w,), jnp.int32),
          pltpu.VMEM((dim, vocab), jnp.float32),
          pltpu.VMEM((n_sub_t, n_lane_t, _SUB, _LANE_T), jnp.float32),
          pltpu.SemaphoreType.DMA,
      ],
      compiler_params=pltpu.CompilerParams(
          use_tc_tiling_on_sc=False, needs_layout_passes=False
      ),
  )
  def emb(tickers_hbm, table_t_hbm, out_hbm, idx_v, tab_v, tr_v, sem):
    wid = lax.axis_index("s") * _NUM_CORES + lax.axis_index("c")
    base = wid * b_per_w
    c_idx = pltpu.async_copy(
        tickers_hbm.at[pl.ds(base, b_per_w)], idx_v, sem
    )
    c_tab = pltpu.async_copy(table_t_hbm, tab_v, sem)
    c_idx.wait()
    c_tab.wait()

    @plsc.parallel_loop(0, b_per_w // _LANES, unroll=4)
    def lookup_group(g):
      j = g // (_LANE_T // _LANES)
      off = (g % (_LANE_T // _LANES)) * _LANES
      t16 = idx_v[pl.ds(g * _LANES, _LANES)]
      for d in range(dim):
        vals = plsc.load_gather(
            tab_v, [jnp.full((_LANES,), d, jnp.int32), t16]
        )
        tr_v[d // _SUB, j, d % _SUB, pl.ds(off, _LANES)] = vals
    for r in range(n_sub_t):
      pltpu.sync_copy(
          tr_v.at[r],
          out_hbm.at[r, pl.ds(wid * n_lane_t, n_lane_t)],
      )

  return emb


def kernel(tickers, table):
  batch = tickers.shape[0]
  vocab, dim = table.shape
  oh = _build(batch, vocab, dim)(tickers, table.T)
  out_t = oh.transpose(0, 2, 1, 3).reshape(dim, batch)
  return out_t.T


# speedup vs baseline: 1.0222x; 1.0222x over previous
import functools

import jax
import jax.numpy as jnp
from jax import lax
from jax.experimental import pallas as pl
from jax.experimental.pallas import tpu as pltpu
from jax.experimental.pallas import tpu_sc as plsc

_NUM_CORES = 2
_NUM_SUBCORES = 16
_NUM_WORKERS = _NUM_CORES * _NUM_SUBCORES
_LANES = 16
_SUB = 8
_LANE_T = 128


@functools.cache
def _build(batch, vocab, dim):
  b_per_w = batch // _NUM_WORKERS
  n_lane_t = b_per_w // _LANE_T
  n_sub_t = dim // _SUB
  mesh = plsc.VectorSubcoreMesh(core_axis_name="c", subcore_axis_name="s")

  @functools.partial(
      pl.kernel,
      mesh=mesh,
      out_type=jax.ShapeDtypeStruct(
          (n_sub_t, batch // _LANE_T, _SUB, _LANE_T), jnp.float32
      ),
      scratch_types=[
          pltpu.VMEM((b_per_w,), jnp.int32),
          pltpu.VMEM((dim, vocab), jnp.float32),
          pltpu.VMEM((n_sub_t, n_lane_t, _SUB, _LANE_T), jnp.float32),
          pltpu.SemaphoreType.DMA,
          pltpu.SemaphoreType.DMA,
          pltpu.SemaphoreType.DMA,
      ],
      compiler_params=pltpu.CompilerParams(
          use_tc_tiling_on_sc=False, needs_layout_passes=False
      ),
  )
  def emb(tickers_hbm, table_t_hbm, out_hbm, idx_v, tab_v, tr_v,
          sem_in, sem_tab2, sem_out):
    wid = lax.axis_index("s") * _NUM_CORES + lax.axis_index("c")
    base = wid * b_per_w
    c_idx = pltpu.async_copy(
        tickers_hbm.at[pl.ds(base, b_per_w)], idx_v, sem_in
    )
    c_tab1 = pltpu.async_copy(
        table_t_hbm.at[pl.ds(0, _SUB)], tab_v.at[pl.ds(0, _SUB)], sem_in
    )
    c_tab2 = pltpu.async_copy(
        table_t_hbm.at[pl.ds(_SUB, _SUB)], tab_v.at[pl.ds(_SUB, _SUB)],
        sem_tab2,
    )
    c_idx.wait()
    c_tab1.wait()

    def gather_pass(r):
      d0 = r * _SUB

      @plsc.parallel_loop(0, b_per_w // _LANES, unroll=2)
      def lookup_group(g):
        j = g // (_LANE_T // _LANES)
        off = (g % (_LANE_T // _LANES)) * _LANES
        t16 = idx_v[pl.ds(g * _LANES, _LANES)]
        for dd in range(_SUB):
          vals = plsc.load_gather(
              tab_v, [jnp.full((_LANES,), d0 + dd, jnp.int32), t16]
          )
          tr_v[r, j, dd, pl.ds(off, _LANES)] = vals

    gather_pass(0)
    c_out0 = pltpu.async_copy(
        tr_v.at[0], out_hbm.at[0, pl.ds(wid * n_lane_t, n_lane_t)], sem_out
    )
    c_tab2.wait()
    gather_pass(1)
    pltpu.sync_copy(
        tr_v.at[1], out_hbm.at[1, pl.ds(wid * n_lane_t, n_lane_t)]
    )
    c_out0.wait()

  return emb


def kernel(tickers, table):
  batch = tickers.shape[0]
  vocab, dim = table.shape
  oh = _build(batch, vocab, dim)(tickers, table.T)
  out_t = oh.transpose(0, 2, 1, 3).reshape(dim, batch)
  return out_t.T
